# baseline (speedup 1.0000x reference)
import functools

import jax
import jax.numpy as jnp
from jax import lax
from jax.experimental import pallas as pl
from jax.experimental.pallas import tpu as pltpu
from jax.experimental.pallas import tpu_sc as plsc

_H = 256


def _sc_deint(offh, offv, ph0, ph1, pv0, pv1, buf_in, buf0, buf1):
    wid = lax.axis_index("s") * 2 + lax.axis_index("c")
    iota = lax.broadcasted_iota(jnp.int32, (16,), 0)

    def do(in_ref, o0, o1, unit):
        p = unit // 2
        hh = unit % 2
        pltpu.sync_copy(in_ref.at[p, pl.ds(hh * 32768, 32768)], buf_in)

        def body(m, carry):
            for c, bufc in ((0, buf0), (1, buf1)):
                for lb in range(16):
                    h128 = 1 if lb >= 8 else 0
                    colc = 2 * ((16 * lb) % 128) + c
                    idx = (2 * m + h128) * 256 + colc + 2 * iota
                    vals = plsc.load_gather(buf_in, [idx])
                    bufc[pl.ds(m * 256 + 16 * lb, 16)] = vals
            return carry

        lax.fori_loop(0, 64, body, 0)
        pltpu.sync_copy(buf0, o0.at[p, pl.ds(hh * 16384, 16384)])
        pltpu.sync_copy(buf1, o1.at[p, pl.ds(hh * 16384, 16384)])

    for step in range(3):
        u = wid + 32 * step

        @pl.when(u < 36)
        def _():
            do(offh, ph0, ph1, u)

        @pl.when(jnp.logical_and(u >= 36, u < 72))
        def _():
            do(offv, pv0, pv1, u - 36)


def _sc_parity_views(offsets_h, offsets_v):
    n = offsets_h.shape[0] * 9
    st = jax.ShapeDtypeStruct((n, 2 * 16384), jnp.float32)
    run = pl.kernel(
        _sc_deint,
        out_type=(st, st, st, st),
        mesh=plsc.VectorSubcoreMesh(core_axis_name="c", subcore_axis_name="s"),
        scratch_types=[pltpu.VMEM((32768,), jnp.float32),
                       pltpu.VMEM((16384,), jnp.float32),
                       pltpu.VMEM((16384,), jnp.float32)],
        compiler_params=pltpu.CompilerParams(needs_layout_passes=False),
    )
    return run(offsets_h.reshape(n, _H * _H), offsets_v.reshape(n, _H * _H))


def _body(oh, ov, ker, imgb, ph0, ph1, pv0, pv1, out):
    first = pl.program_id(1) == 0

    lint = jax.lax.broadcasted_iota(jnp.int32, (1, _H), 1)
    jlane = lint.astype(jnp.float32)
    jp = (2 * (lint % 128)).astype(jnp.float32)

    nr, nc = imgb.shape[2], imgb.shape[3]
    r_io = jax.lax.broadcasted_iota(jnp.int32, (nr, nc), 0)
    l_io = jax.lax.broadcasted_iota(jnp.int32, (nr, nc), 1)
    diag = {}
    for s in range(-3, 4):
        mask = (l_io - r_io) == s
        for cch in range(3):
            M = imgb[0, cch]
            bd = jnp.sum(jnp.where(mask, M, 0.0), axis=0, keepdims=True)
            for a in range(4):
                b2 = a + s
                if 0 <= b2 <= 3:
                    diag[(a, b2, cch)] = bd[:, b2:b2 + _H]

    def srcw(p_refs, k, t, is_x):
        q = 2 * k + t
        c, kp = q // 9, q % 9
        add = kp // 3 if is_x else kp % 3
        x = (p_refs[c][0, kp] + 1.5) + add
        x = x + (jp + (c + 0.5))
        fl = jnp.floor(x)
        return jnp.where(first, (fl + 1.0) - x, x - fl)

    acc0 = acc1 = acc2 = None
    for k in range(9):
        rk, ck = k // 3, k % 3
        xs = ((oh[0, k] + 1.5) + rk) + (jlane + 0.5)
        ys = ((ov[0, k] + 1.5) + ck) + (jlane + 0.5)
        bx = jnp.floor(xs) - (jlane + (rk + 2))
        by = jnp.floor(ys) - (jlane + (ck + 2))
        w0 = srcw((ph0, ph1), k, 0, True)
        w1 = srcw((ph0, ph1), k, 1, True)
        v0 = srcw((pv0, pv1), k, 0, False)
        v1 = srcw((pv0, pv1), k, 1, False)
        g = []
        for cch in range(3):
            v00 = diag[(rk, ck, cch)]
            v01 = diag[(rk, ck + 1, cch)]
            v10 = diag[(rk + 1, ck, cch)]
            v11 = diag[(rk + 1, ck + 1, cch)]
            g.append((1 - bx) * ((1 - by) * v00 + by * v01)
                     + bx * ((1 - by) * v10 + by * v11))
        g0, g1, g2 = g
        kv = ker[0, k]
        r0 = v0 * (w0 * g0 + w1 * g0) + v1 * (w0 * g1 + w1 * g2)
        r1 = v0 * (w0 * g0 + w1 * g1) + v1 * (w0 * g1 + w1 * g2)
        r2 = v0 * (w0 * g0 + w1 * g1) + v1 * (w0 * g2 + w1 * g2)
        if acc0 is None:
            acc0, acc1, acc2 = kv * r0, kv * r1, kv * r2
        else:
            acc0, acc1, acc2 = acc0 + kv * r0, acc1 + kv * r1, acc2 + kv * r2

    for cch, acc in enumerate((acc0, acc1, acc2)):
        o = acc * 255.0
        out[0, cch] = o - jnp.sin(2 * jnp.pi * o) / (2 * jnp.pi)


def kernel(img, kernels, offsets_h, offsets_v):
    B = img.shape[0]
    imgb = img[:, :, 2:262, 2:266]
    ph0, ph1, pv0, pv1 = _sc_parity_views(offsets_h, offsets_v)
    ph0 = ph0.reshape(B, 9, 128, _H)
    ph1 = ph1.reshape(B, 9, 128, _H)
    pv0 = pv0.reshape(B, 9, 128, _H)
    pv1 = pv1.reshape(B, 9, 128, _H)

    half = pl.BlockSpec((1, 9, 128, _H), lambda b, h: (b, 0, h, 0))
    full = pl.BlockSpec((1, 9, 128, _H), lambda b, h: (b, 0, 0, 0))
    out = pl.pallas_call(
        _body,
        grid=(B, 2),
        in_specs=[half, half, half,
                  pl.BlockSpec((1, 3, 260, 264), lambda b, h: (b, 0, 0, 0)),
                  full, full, full, full],
        out_specs=pl.BlockSpec((1, 3, 128, _H), lambda b, h: (b, 0, h, 0)),
        out_shape=jax.ShapeDtypeStruct((B, 3, _H, _H), jnp.float32),
    )(offsets_h, offsets_v, kernels, imgb, ph0, ph1, pv0, pv1)
    return jnp.transpose(out, (0, 2, 3, 1))


# speedup vs baseline: 1.5632x; 1.5632x over previous
import functools

import jax
import jax.numpy as jnp
from jax import lax
from jax.experimental import pallas as pl
from jax.experimental.pallas import tpu as pltpu
from jax.experimental.pallas import tpu_sc as plsc

_H = 256


def _parity_views(offsets_h, offsets_v):
    B = offsets_h.shape[0]
    li = jnp.arange(_H)[:, None]
    jj = jnp.arange(128)[None, :]
    outs = []
    for offs in (offsets_h, offsets_v):
        flat = offs.reshape(B * 9 * _H, _H)
        for c in (0, 1):
            sel = (li == 2 * jj + c).astype(jnp.float32)
            d = jax.lax.dot_general(flat, sel, (((1,), (0,)), ((), ())),
                                    precision=jax.lax.Precision.HIGHEST)
            outs.append(d.reshape(B, 9, 128, _H))
    return outs


def _body(oh, ov, ker, imgb, ph0, ph1, pv0, pv1, out):
    first = pl.program_id(1) == 0

    lint = jax.lax.broadcasted_iota(jnp.int32, (1, _H), 1)
    jlane = lint.astype(jnp.float32)
    jp = (2 * (lint % 128)).astype(jnp.float32)

    nr, nc = imgb.shape[2], imgb.shape[3]
    r_io = jax.lax.broadcasted_iota(jnp.int32, (nr, nc), 0)
    l_io = jax.lax.broadcasted_iota(jnp.int32, (nr, nc), 1)
    diag = {}
    for s in range(-3, 4):
        mask = (l_io - r_io) == s
        for cch in range(3):
            M = imgb[0, cch]
            bd = jnp.sum(jnp.where(mask, M, 0.0), axis=0, keepdims=True)
            for a in range(4):
                b2 = a + s
                if 0 <= b2 <= 3:
                    diag[(a, b2, cch)] = bd[:, b2:b2 + _H]

    def srcw(p_refs, k, t, is_x):
        q = 2 * k + t
        c, kp = q // 9, q % 9
        add = kp // 3 if is_x else kp % 3
        x = (p_refs[c][0, kp] + 1.5) + add
        x = x + (jp + (c + 0.5))
        fl = jnp.floor(x)
        return jnp.where(first, (fl + 1.0) - x, x - fl)

    acc0 = acc1 = acc2 = None
    for k in range(9):
        rk, ck = k // 3, k % 3
        xs = ((oh[0, k] + 1.5) + rk) + (jlane + 0.5)
        ys = ((ov[0, k] + 1.5) + ck) + (jlane + 0.5)
        bx = jnp.floor(xs) - (jlane + (rk + 2))
        by = jnp.floor(ys) - (jlane + (ck + 2))
        w0 = srcw((ph0, ph1), k, 0, True)
        w1 = srcw((ph0, ph1), k, 1, True)
        v0 = srcw((pv0, pv1), k, 0, False)
        v1 = srcw((pv0, pv1), k, 1, False)
        g = []
        for cch in range(3):
            v00 = diag[(rk, ck, cch)]
            v01 = diag[(rk, ck + 1, cch)]
            v10 = diag[(rk + 1, ck, cch)]
            v11 = diag[(rk + 1, ck + 1, cch)]
            g.append((1 - bx) * ((1 - by) * v00 + by * v01)
                     + bx * ((1 - by) * v10 + by * v11))
        g0, g1, g2 = g
        kv = ker[0, k]
        r0 = v0 * (w0 * g0 + w1 * g0) + v1 * (w0 * g1 + w1 * g2)
        r1 = v0 * (w0 * g0 + w1 * g1) + v1 * (w0 * g1 + w1 * g2)
        r2 = v0 * (w0 * g0 + w1 * g1) + v1 * (w0 * g2 + w1 * g2)
        if acc0 is None:
            acc0, acc1, acc2 = kv * r0, kv * r1, kv * r2
        else:
            acc0, acc1, acc2 = acc0 + kv * r0, acc1 + kv * r1, acc2 + kv * r2

    for cch, acc in enumerate((acc0, acc1, acc2)):
        o = acc * 255.0
        out[0, cch] = o - jnp.sin(2 * jnp.pi * o) / (2 * jnp.pi)


def kernel(img, kernels, offsets_h, offsets_v):
    B = img.shape[0]
    imgb = img[:, :, 2:262, 2:266]
    ph0, ph1, pv0, pv1 = _parity_views(offsets_h, offsets_v)

    half = pl.BlockSpec((1, 9, 128, _H), lambda b, h: (b, 0, h, 0))
    full = pl.BlockSpec((1, 9, 128, _H), lambda b, h: (b, 0, 0, 0))
    out = pl.pallas_call(
        _body,
        grid=(B, 2),
        in_specs=[half, half, half,
                  pl.BlockSpec((1, 3, 260, 264), lambda b, h: (b, 0, 0, 0)),
                  full, full, full, full],
        out_specs=pl.BlockSpec((1, 3, 128, _H), lambda b, h: (b, 0, h, 0)),
        out_shape=jax.ShapeDtypeStruct((B, 3, _H, _H), jnp.float32),
    )(offsets_h, offsets_v, kernels, imgb, ph0, ph1, pv0, pv1)
    return jnp.transpose(out, (0, 2, 3, 1))


# speedup vs baseline: 2.5986x; 1.6623x over previous
import jax
import jax.numpy as jnp
from jax import lax
from jax.experimental import pallas as pl

_H = 256


def _body(oh, ov, ker, imgb, out):
    pid = pl.program_id(1)
    first = pid == 0

    lint = jax.lax.broadcasted_iota(jnp.int32, (1, _H), 1)
    jlane = lint.astype(jnp.float32)
    jp = (2 * (lint % 128)).astype(jnp.float32)

    li = jax.lax.broadcasted_iota(jnp.int32, (_H, _H), 0)
    co = jax.lax.broadcasted_iota(jnp.int32, (_H, _H), 1)
    sel = (li == 2 * (co % 128) + co // 128).astype(jnp.float32)

    nr, nc = imgb.shape[2], imgb.shape[3]
    r_io = jax.lax.broadcasted_iota(jnp.int32, (nr, nc), 0)
    l_io = jax.lax.broadcasted_iota(jnp.int32, (nr, nc), 1)
    diag = {}
    for s in range(-3, 4):
        mask = (l_io - r_io) == s
        for cch in range(3):
            M = imgb[0, cch]
            bd = jnp.sum(jnp.where(mask, M, 0.0), axis=0, keepdims=True)
            for a in range(4):
                b2 = a + s
                if 0 <= b2 <= 3:
                    diag[(a, b2, cch)] = bd[:, b2:b2 + _H]

    dh = [None] * 9
    dv = [None] * 9
    for kp in range(9):
        dh[kp] = lax.dot_general(oh[0, kp], sel, (((1,), (0,)), ((), ())),
                                 precision=lax.Precision.HIGHEST)
        dv[kp] = lax.dot_general(ov[0, kp], sel, (((1,), (0,)), ((), ())),
                                 precision=lax.Precision.HIGHEST)

    def srcw(dlist, k, t, is_x):
        q = 2 * k + t
        c, kp = q // 9, q % 9
        add = kp // 3 if is_x else kp % 3
        dc = dlist[kp][:, 128 * c:128 * c + 128]
        r3 = dc.reshape(128, 2, 128)
        src = jnp.concatenate([r3[:, 0, :], r3[:, 1, :]], axis=1)
        x = (src + 1.5) + add
        x = x + (jp + (c + 0.5))
        fl = jnp.floor(x)
        return jnp.where(first, (fl + 1.0) - x, x - fl)

    row0 = pid * 128
    acc0 = acc1 = acc2 = None
    for k in range(9):
        rk, ck = k // 3, k % 3
        ohk = oh[0, k, pl.ds(row0, 128), :]
        ovk = ov[0, k, pl.ds(row0, 128), :]
        xs = ((ohk + 1.5) + rk) + (jlane + 0.5)
        ys = ((ovk + 1.5) + ck) + (jlane + 0.5)
        bx = jnp.floor(xs) - (jlane + (rk + 2))
        by = jnp.floor(ys) - (jlane + (ck + 2))
        w0 = srcw(dh, k, 0, True)
        w1 = srcw(dh, k, 1, True)
        v0 = srcw(dv, k, 0, False)
        v1 = srcw(dv, k, 1, False)
        g = []
        for cch in range(3):
            v00 = diag[(rk, ck, cch)]
            v01 = diag[(rk, ck + 1, cch)]
            v10 = diag[(rk + 1, ck, cch)]
            v11 = diag[(rk + 1, ck + 1, cch)]
            g.append((1 - bx) * ((1 - by) * v00 + by * v01)
                     + bx * ((1 - by) * v10 + by * v11))
        g0, g1, g2 = g
        kv = ker[0, k]
        r0 = v0 * (w0 * g0 + w1 * g0) + v1 * (w0 * g1 + w1 * g2)
        r1 = v0 * (w0 * g0 + w1 * g1) + v1 * (w0 * g1 + w1 * g2)
        r2 = v0 * (w0 * g0 + w1 * g1) + v1 * (w0 * g2 + w1 * g2)
        if acc0 is None:
            acc0, acc1, acc2 = kv * r0, kv * r1, kv * r2
        else:
            acc0, acc1, acc2 = acc0 + kv * r0, acc1 + kv * r1, acc2 + kv * r2

    for cch, acc in enumerate((acc0, acc1, acc2)):
        o = acc * 255.0
        out[0, cch] = o - jnp.sin(2 * jnp.pi * o) / (2 * jnp.pi)


def kernel(img, kernels, offsets_h, offsets_v):
    B = img.shape[0]
    imgb = img[:, :, 2:262, 2:266]

    full = pl.BlockSpec((1, 9, _H, _H), lambda b, h: (b, 0, 0, 0))
    half = pl.BlockSpec((1, 9, 128, _H), lambda b, h: (b, 0, h, 0))
    out = pl.pallas_call(
        _body,
        grid=(B, 2),
        in_specs=[full, full, half,
                  pl.BlockSpec((1, 3, 260, 264), lambda b, h: (b, 0, 0, 0))],
        out_specs=pl.BlockSpec((1, 3, 128, _H), lambda b, h: (b, 0, h, 0)),
        out_shape=jax.ShapeDtypeStruct((B, 3, _H, _H), jnp.float32),
    )(offsets_h, offsets_v, kernels, imgb)
    return jnp.transpose(out, (0, 2, 3, 1))


# speedup vs baseline: 3.5324x; 1.3593x over previous
import jax
import jax.numpy as jnp
from jax import lax
from jax.experimental import pallas as pl

_H = 256


def _body(oh, ov, ker, imgb, out):
    lint = jax.lax.broadcasted_iota(jnp.int32, (1, _H), 1)
    jlane = lint.astype(jnp.float32)
    jp = (2 * (lint % 128)).astype(jnp.float32)

    li = jax.lax.broadcasted_iota(jnp.int32, (_H, _H), 0)
    co = jax.lax.broadcasted_iota(jnp.int32, (_H, _H), 1)
    sel = (li == 2 * (co % 128) + co // 128).astype(jnp.float32)

    nr, nc = imgb.shape[2], imgb.shape[3]
    r_io = jax.lax.broadcasted_iota(jnp.int32, (nr, nc), 0)
    l_io = jax.lax.broadcasted_iota(jnp.int32, (nr, nc), 1)
    diag = {}
    for s in range(-3, 4):
        mask = (l_io - r_io) == s
        for cch in range(3):
            M = imgb[0, cch]
            bd = jnp.sum(jnp.where(mask, M, 0.0), axis=0, keepdims=True)
            for a in range(4):
                b2 = a + s
                if 0 <= b2 <= 3:
                    diag[(a, b2, cch)] = bd[:, b2:b2 + _H]

    dh = [None] * 9
    dv = [None] * 9
    for kp in range(9):
        dh[kp] = lax.dot_general(oh[0, kp], sel, (((1,), (0,)), ((), ())),
                                 precision=lax.Precision.HIGHEST)
        dv[kp] = lax.dot_general(ov[0, kp], sel, (((1,), (0,)), ((), ())),
                                 precision=lax.Precision.HIGHEST)

    def srcw(dlist, k, t, is_x):
        q = 2 * k + t
        c, kp = q // 9, q % 9
        add = kp // 3 if is_x else kp % 3
        dc = dlist[kp][:, 128 * c:128 * c + 128]
        r3 = dc.reshape(128, 2, 128)
        src = jnp.concatenate([r3[:, 0, :], r3[:, 1, :]], axis=1)
        x = (src + 1.5) + add
        x = x + (jp + (c + 0.5))
        fl = jnp.floor(x)
        return jnp.concatenate([(fl + 1.0) - x, x - fl], axis=0)

    acc0 = acc1 = acc2 = None
    for k in range(9):
        rk, ck = k // 3, k % 3
        xs = ((oh[0, k] + 1.5) + rk) + (jlane + 0.5)
        ys = ((ov[0, k] + 1.5) + ck) + (jlane + 0.5)
        bx = jnp.floor(xs) - (jlane + (rk + 2))
        by = jnp.floor(ys) - (jlane + (ck + 2))
        w0 = srcw(dh, k, 0, True)
        w1 = srcw(dh, k, 1, True)
        v0 = srcw(dv, k, 0, False)
        v1 = srcw(dv, k, 1, False)
        g = []
        for cch in range(3):
            v00 = diag[(rk, ck, cch)]
            v01 = diag[(rk, ck + 1, cch)]
            v10 = diag[(rk + 1, ck, cch)]
            v11 = diag[(rk + 1, ck + 1, cch)]
            g.append((1 - bx) * ((1 - by) * v00 + by * v01)
                     + bx * ((1 - by) * v10 + by * v11))
        g0, g1, g2 = g
        kv = ker[0, k]
        r0 = v0 * (w0 * g0 + w1 * g0) + v1 * (w0 * g1 + w1 * g2)
        r1 = v0 * (w0 * g0 + w1 * g1) + v1 * (w0 * g1 + w1 * g2)
        r2 = v0 * (w0 * g0 + w1 * g1) + v1 * (w0 * g2 + w1 * g2)
        if acc0 is None:
            acc0, acc1, acc2 = kv * r0, kv * r1, kv * r2
        else:
            acc0, acc1, acc2 = acc0 + kv * r0, acc1 + kv * r1, acc2 + kv * r2

    for cch, acc in enumerate((acc0, acc1, acc2)):
        o = acc * 255.0
        out[0, cch] = o - jnp.sin(2 * jnp.pi * o) / (2 * jnp.pi)


def kernel(img, kernels, offsets_h, offsets_v):
    B = img.shape[0]
    imgb = img[:, :, 2:262, 2:266]

    full = pl.BlockSpec((1, 9, _H, _H), lambda b: (b, 0, 0, 0))
    out = pl.pallas_call(
        _body,
        grid=(B,),
        in_specs=[full, full, full,
                  pl.BlockSpec((1, 3, 260, 264), lambda b: (b, 0, 0, 0))],
        out_specs=pl.BlockSpec((1, 3, _H, _H), lambda b: (b, 0, 0, 0)),
        out_shape=jax.ShapeDtypeStruct((B, 3, _H, _H), jnp.float32),
    )(offsets_h, offsets_v, kernels, imgb)
    return jnp.transpose(out, (0, 2, 3, 1))


# speedup vs baseline: 3.6636x; 1.0372x over previous
import jax
import jax.numpy as jnp
from jax import lax
from jax.experimental import pallas as pl

_H = 256


def _body(oh, ov, ker, imgb, out):
    lint = jax.lax.broadcasted_iota(jnp.int32, (1, _H), 1)
    jlane = lint.astype(jnp.float32)
    jp = (2 * (lint % 128)).astype(jnp.float32)

    li = jax.lax.broadcasted_iota(jnp.int32, (_H, _H), 0)
    co = jax.lax.broadcasted_iota(jnp.int32, (_H, _H), 1)
    sel = (li == 2 * (co % 128) + co // 128).astype(jnp.float32)

    nr, nc = imgb.shape[2], imgb.shape[3]
    r_io = jax.lax.broadcasted_iota(jnp.int32, (nr, nc), 0)
    l_io = jax.lax.broadcasted_iota(jnp.int32, (nr, nc), 1)
    diag = {}
    for s in range(-3, 4):
        mask = (l_io - r_io) == s
        for cch in range(3):
            M = imgb[0, cch]
            bd = jnp.sum(jnp.where(mask, M, 0.0), axis=0, keepdims=True)
            for a in range(4):
                b2 = a + s
                if 0 <= b2 <= 3:
                    diag[(a, b2, cch)] = bd[:, b2:b2 + _H]

    selb = sel.astype(jnp.bfloat16)
    dn = (((1,), (0,)), ((), ()))

    def deint(x):
        h1 = x.astype(jnp.bfloat16)
        r1 = x - h1.astype(jnp.float32)
        h2 = r1.astype(jnp.bfloat16)
        h3 = (r1 - h2.astype(jnp.float32)).astype(jnp.bfloat16)
        d = lax.dot_general(h1, selb, dn, preferred_element_type=jnp.float32)
        d = d + lax.dot_general(h2, selb, dn, preferred_element_type=jnp.float32)
        return d + lax.dot_general(h3, selb, dn, preferred_element_type=jnp.float32)

    bxl, byl, dh, dv = [], [], [], []
    for k in range(9):
        rk, ck = k // 3, k % 3
        xs = ((oh[0, k] + 1.5) + rk) + (jlane + 0.5)
        ys = ((ov[0, k] + 1.5) + ck) + (jlane + 0.5)
        flx, fly = jnp.floor(xs), jnp.floor(ys)
        bxl.append(flx - (jlane + (rk + 2)))
        byl.append(fly - (jlane + (ck + 2)))
        dh.append(deint(xs - flx))
        dv.append(deint(ys - fly))

    def srcw(dlist, k, t):
        q = 2 * k + t
        c, kp = q // 9, q % 9
        dc = dlist[kp][:, 128 * c:128 * c + 128]
        r3 = dc.reshape(128, 2, 128)
        src = jnp.concatenate([r3[:, 0, :], r3[:, 1, :]], axis=1)
        return jnp.concatenate([1.0 - src, src], axis=0)

    acc0 = acc1 = acc2 = None
    for k in range(9):
        rk, ck = k // 3, k % 3
        bx, by = bxl[k], byl[k]
        w0 = srcw(dh, k, 0)
        w1 = srcw(dh, k, 1)
        v0 = srcw(dv, k, 0)
        v1 = srcw(dv, k, 1)
        g = []
        for cch in range(3):
            v00 = diag[(rk, ck, cch)]
            v01 = diag[(rk, ck + 1, cch)]
            v10 = diag[(rk + 1, ck, cch)]
            v11 = diag[(rk + 1, ck + 1, cch)]
            g.append((1 - bx) * ((1 - by) * v00 + by * v01)
                     + bx * ((1 - by) * v10 + by * v11))
        g0, g1, g2 = g
        kv = ker[0, k]
        r0 = v0 * (w0 * g0 + w1 * g0) + v1 * (w0 * g1 + w1 * g2)
        r1 = v0 * (w0 * g0 + w1 * g1) + v1 * (w0 * g1 + w1 * g2)
        r2 = v0 * (w0 * g0 + w1 * g1) + v1 * (w0 * g2 + w1 * g2)
        if acc0 is None:
            acc0, acc1, acc2 = kv * r0, kv * r1, kv * r2
        else:
            acc0, acc1, acc2 = acc0 + kv * r0, acc1 + kv * r1, acc2 + kv * r2

    for cch, acc in enumerate((acc0, acc1, acc2)):
        o = acc * 255.0
        out[0, cch] = o - jnp.sin(2 * jnp.pi * o) / (2 * jnp.pi)


def kernel(img, kernels, offsets_h, offsets_v):
    B = img.shape[0]
    imgb = img[:, :, 2:262, 2:266]

    full = pl.BlockSpec((1, 9, _H, _H), lambda b: (b, 0, 0, 0))
    out = pl.pallas_call(
        _body,
        grid=(B,),
        in_specs=[full, full, full,
                  pl.BlockSpec((1, 3, 260, 264), lambda b: (b, 0, 0, 0))],
        out_specs=pl.BlockSpec((1, 3, _H, _H), lambda b: (b, 0, 0, 0)),
        out_shape=jax.ShapeDtypeStruct((B, 3, _H, _H), jnp.float32),
    )(offsets_h, offsets_v, kernels, imgb)
    return jnp.transpose(out, (0, 2, 3, 1))
